# baseline (speedup 1.0000x reference)
import functools

import jax
import jax.numpy as jnp
from jax import lax
from jax.experimental import pallas as pl
from jax.experimental.pallas import tpu as pltpu
from jax.experimental.pallas import tpu_sc as plsc

N_NODES = 10000
N_PAD = 10240
E_EDGES = 320000
E_PAD = 327680
PAD_NODE = 10200
NC, NS = 2, 16
RPT = N_PAD // NS
CHUNK = 128
NCH = E_PAD // (NC * NS * CHUNK)
BLK = 512
GRID = N_PAD // BLK

_mesh = plsc.VectorSubcoreMesh(core_axis_name="c", subcore_axis_name="s")


@functools.partial(
    pl.kernel,
    mesh=_mesh,
    out_type=jax.ShapeDtypeStruct((NC * N_PAD, 128), jnp.float32),
    scratch_types=[
        pltpu.VMEM((NCH, CHUNK), jnp.int32),
        pltpu.VMEM((CHUNK, 128), jnp.float32),
        pltpu.VMEM_SHARED((N_PAD, 128), jnp.float32),
    ],
)
def _deg_kernel(dst_hbm, ones_hbm, zero_hbm, deg_out, idx_v, ones_v, deg_sh):
    c = lax.axis_index("c")
    s = lax.axis_index("s")
    w = c * NS + s
    pltpu.sync_copy(dst_hbm.at[w], idx_v)
    pltpu.sync_copy(ones_hbm, ones_v)
    pltpu.sync_copy(zero_hbm, deg_sh.at[pl.ds(s * RPT, RPT)])
    plsc.subcore_barrier()

    def body(j, carry):
        pltpu.sync_copy(ones_v, deg_sh.at[idx_v.at[j]], add=True)
        return carry

    lax.fori_loop(0, NCH, body, 0)
    plsc.subcore_barrier()
    pltpu.sync_copy(deg_sh.at[pl.ds(s * RPT, RPT)],
                    deg_out.at[pl.ds(c * N_PAD + s * RPT, RPT)])


@functools.partial(
    pl.kernel,
    mesh=_mesh,
    out_type=jax.ShapeDtypeStruct((NC * N_PAD, 128), jnp.float32),
    scratch_types=[
        pltpu.VMEM((NCH, CHUNK), jnp.int32),
        pltpu.VMEM((NCH, CHUNK), jnp.int32),
        pltpu.VMEM((CHUNK, 128), jnp.float32),
        pltpu.VMEM_SHARED((N_PAD, 128), jnp.float32),
        pltpu.SemaphoreType.DMA,
    ],
)
def _prop(g_hbm, zero_hbm, src_hbm, dst_hbm, s_out,
          src_v, dst_v, rows_v, acc_sh, gsem):
    c = lax.axis_index("c")
    s = lax.axis_index("s")
    w = c * NS + s
    pltpu.sync_copy(src_hbm.at[w], src_v)
    pltpu.sync_copy(dst_hbm.at[w], dst_v)

    @pl.when(c == 0)
    def _():
        pltpu.sync_copy(g_hbm.at[pl.ds(s * RPT, RPT)],
                        acc_sh.at[pl.ds(s * RPT, RPT)])

    @pl.when(c == 1)
    def _():
        pltpu.sync_copy(zero_hbm, acc_sh.at[pl.ds(s * RPT, RPT)])

    plsc.subcore_barrier()

    def body(j, carry):
        pltpu.async_copy(g_hbm.at[src_v.at[j]], rows_v, gsem).wait()
        pltpu.sync_copy(rows_v, acc_sh.at[dst_v.at[j]], add=True)
        return carry

    lax.fori_loop(0, NCH, body, 0)
    plsc.subcore_barrier()
    pltpu.sync_copy(acc_sh.at[pl.ds(s * RPT, RPT)],
                    s_out.at[pl.ds(c * N_PAD + s * RPT, RPT)])


def _tc1_body(x_ref, w_ref, deg_ref, g_ref):
    deg = deg_ref[0, :, 0] + deg_ref[1, :, 0] + 1.0
    dinv = lax.rsqrt(deg)
    h = jnp.dot(x_ref[...], w_ref[...], preferred_element_type=jnp.float32)
    g_ref[...] = h * dinv[:, None]


def _tc1(x, W1, deg):
    return pl.pallas_call(
        _tc1_body,
        grid=(GRID,),
        in_specs=[
            pl.BlockSpec((BLK, 128), lambda i: (i, 0)),
            pl.BlockSpec((128, 128), lambda i: (0, 0)),
            pl.BlockSpec((2, BLK, 8), lambda i: (0, i, 0)),
        ],
        out_specs=pl.BlockSpec((BLK, 128), lambda i: (i, 0)),
        out_shape=jax.ShapeDtypeStruct((N_PAD, 128), jnp.float32),
    )(x, W1, deg)


def _tc2_body(s_ref, w_ref, b_ref, deg_ref, g_ref):
    deg = deg_ref[0, :, 0] + deg_ref[1, :, 0] + 1.0
    dinv = lax.rsqrt(deg)
    x2 = (s_ref[0] + s_ref[1]) * dinv[:, None] + b_ref[0][None, :]
    h = jnp.dot(x2, w_ref[...], preferred_element_type=jnp.float32)
    g = h * dinv[:, None]
    g_ref[...] = jnp.concatenate([g, jnp.zeros_like(g)], axis=1)


def _tc2(s1, W2, b1, deg):
    return pl.pallas_call(
        _tc2_body,
        grid=(GRID,),
        in_specs=[
            pl.BlockSpec((2, BLK, 128), lambda i: (0, i, 0)),
            pl.BlockSpec((128, 64), lambda i: (0, 0)),
            pl.BlockSpec((1, 128), lambda i: (0, 0)),
            pl.BlockSpec((2, BLK, 8), lambda i: (0, i, 0)),
        ],
        out_specs=pl.BlockSpec((BLK, 128), lambda i: (i, 0)),
        out_shape=jax.ShapeDtypeStruct((N_PAD, 128), jnp.float32),
    )(s1, W2, b1, deg)


def _tc3_body(s_ref, b_ref, deg_ref, z_ref):
    deg = deg_ref[0, :, 0] + deg_ref[1, :, 0] + 1.0
    dinv = lax.rsqrt(deg)
    z = (s_ref[0, :, :64] + s_ref[1, :, :64]) * dinv[:, None]
    z_ref[...] = z + b_ref[0][None, :]


def _tc3(s2, b2, deg):
    return pl.pallas_call(
        _tc3_body,
        grid=(GRID,),
        in_specs=[
            pl.BlockSpec((2, BLK, 128), lambda i: (0, i, 0)),
            pl.BlockSpec((1, 64), lambda i: (0, 0)),
            pl.BlockSpec((2, BLK, 8), lambda i: (0, i, 0)),
        ],
        out_specs=pl.BlockSpec((BLK, 64), lambda i: (i, 0)),
        out_shape=jax.ShapeDtypeStruct((N_PAD, 64), jnp.float32),
    )(s2, b2, deg)


def kernel(x, edge_index, W1, b1, W2, b2):
    ei = edge_index.astype(jnp.int32)
    pad = E_PAD - E_EDGES
    src = jnp.concatenate([ei[0], jnp.full((pad,), PAD_NODE, jnp.int32)])
    dst = jnp.concatenate([ei[1], jnp.full((pad,), PAD_NODE, jnp.int32)])
    src32 = src.reshape(NC * NS, NCH, CHUNK)
    dst32 = dst.reshape(NC * NS, NCH, CHUNK)
    dst_deg = dst.reshape(NC * NS, NCH, CHUNK)

    ones128 = jnp.ones((CHUNK, 128), jnp.float32)
    zero128 = jnp.zeros((RPT, 128), jnp.float32)
    x_pad = jnp.pad(x, ((0, N_PAD - N_NODES), (0, 0)))

    deg_parts = _deg_kernel(dst_deg, ones128, zero128)
    deg = deg_parts.reshape(NC, N_PAD, 128)[:, :, :8]

    g1 = _tc1(x_pad, W1, deg)
    s1 = _prop(g1, zero128, src32, dst32).reshape(NC, N_PAD, 128)

    g2 = _tc2(s1, W2, b1.reshape(1, 128), deg)
    s2 = _prop(g2, zero128, src32, dst32).reshape(NC, N_PAD, 128)

    z = _tc3(s2, b2.reshape(1, 64), deg)
    return z[:N_NODES]


# speedup vs baseline: 2.3356x; 2.3356x over previous
import functools

import jax
import jax.numpy as jnp
from jax import lax
from jax.experimental import pallas as pl
from jax.experimental.pallas import tpu as pltpu
from jax.experimental.pallas import tpu_sc as plsc

N_NODES = 10000
N_PAD = 10240
E_EDGES = 320000
E_PAD = 327680
PAD_NODE = 10200
NC, NS = 2, 16
RPT = N_PAD // NS
CHUNK = 128
NCH = E_PAD // (NC * NS * CHUNK)
BLK = 512
GRID = N_PAD // BLK

_mesh = plsc.VectorSubcoreMesh(core_axis_name="c", subcore_axis_name="s")


@functools.partial(
    pl.kernel,
    mesh=_mesh,
    out_type=jax.ShapeDtypeStruct((NC * N_PAD, 128), jnp.float32),
    scratch_types=[
        pltpu.VMEM((NCH, CHUNK), jnp.int32),
        pltpu.VMEM((CHUNK, 128), jnp.float32),
        pltpu.VMEM_SHARED((N_PAD, 128), jnp.float32),
    ],
)
def _deg_kernel(dst_hbm, ones_hbm, zero_hbm, deg_out, idx_v, ones_v, deg_sh):
    c = lax.axis_index("c")
    s = lax.axis_index("s")
    w = c * NS + s
    pltpu.sync_copy(dst_hbm.at[w], idx_v)
    pltpu.sync_copy(ones_hbm, ones_v)
    pltpu.sync_copy(zero_hbm, deg_sh.at[pl.ds(s * RPT, RPT)])
    plsc.subcore_barrier()

    def body(j, carry):
        pltpu.sync_copy(ones_v, deg_sh.at[idx_v.at[j]], add=True)
        return carry

    lax.fori_loop(0, NCH, body, 0)
    plsc.subcore_barrier()
    pltpu.sync_copy(deg_sh.at[pl.ds(s * RPT, RPT)],
                    deg_out.at[pl.ds(c * N_PAD + s * RPT, RPT)])


@functools.partial(
    pl.kernel,
    mesh=_mesh,
    out_type=jax.ShapeDtypeStruct((NC * N_PAD, 128), jnp.float32),
    scratch_types=[
        pltpu.VMEM((NCH, CHUNK), jnp.int32),
        pltpu.VMEM((NCH, CHUNK), jnp.int32),
        pltpu.VMEM((CHUNK, 128), jnp.float32),
        pltpu.VMEM_SHARED((N_PAD, 128), jnp.float32),
        pltpu.SemaphoreType.DMA,
    ],
)
def _prop(g_hbm, zero_hbm, src_hbm, dst_hbm, s_out,
          src_v, dst_v, rows_v, acc_sh, gsem):
    c = lax.axis_index("c")
    s = lax.axis_index("s")
    w = c * NS + s
    pltpu.sync_copy(src_hbm.at[w], src_v)
    pltpu.sync_copy(dst_hbm.at[w], dst_v)

    @pl.when(c == 0)
    def _():
        pltpu.sync_copy(g_hbm.at[pl.ds(s * RPT, RPT)],
                        acc_sh.at[pl.ds(s * RPT, RPT)])

    @pl.when(c == 1)
    def _():
        pltpu.sync_copy(zero_hbm, acc_sh.at[pl.ds(s * RPT, RPT)])

    plsc.subcore_barrier()

    def body(j, carry):
        pltpu.async_copy(g_hbm.at[src_v.at[j]], rows_v, gsem).wait()
        pltpu.sync_copy(rows_v, acc_sh.at[dst_v.at[j]], add=True)
        return carry

    lax.fori_loop(0, NCH, body, 0)
    plsc.subcore_barrier()
    pltpu.sync_copy(acc_sh.at[pl.ds(s * RPT, RPT)],
                    s_out.at[pl.ds(c * N_PAD + s * RPT, RPT)])


def _tc1_body(x_ref, w_ref, deg_ref, g_ref):
    deg = deg_ref[0, :, 0] + deg_ref[1, :, 0] + 1.0
    dinv = lax.rsqrt(deg)
    h = jnp.dot(x_ref[...], w_ref[...], preferred_element_type=jnp.float32)
    g_ref[...] = h * dinv[:, None]


def _tc1(x, W1, deg):
    return pl.pallas_call(
        _tc1_body,
        grid=(GRID,),
        in_specs=[
            pl.BlockSpec((BLK, 128), lambda i: (i, 0)),
            pl.BlockSpec((128, 128), lambda i: (0, 0)),
            pl.BlockSpec((2, BLK, 8), lambda i: (0, i, 0)),
        ],
        out_specs=pl.BlockSpec((BLK, 128), lambda i: (i, 0)),
        out_shape=jax.ShapeDtypeStruct((N_PAD, 128), jnp.float32),
    )(x, W1, deg)


def _tc2_body(s_ref, w_ref, b_ref, deg_ref, g_ref):
    deg = deg_ref[0, :, 0] + deg_ref[1, :, 0] + 1.0
    dinv = lax.rsqrt(deg)
    x2 = (s_ref[0] + s_ref[1]) * dinv[:, None] + b_ref[0][None, :]
    h = jnp.dot(x2, w_ref[...], preferred_element_type=jnp.float32)
    g = h * dinv[:, None]
    g_ref[...] = jnp.concatenate([g, jnp.zeros_like(g)], axis=1)


def _tc2(s1, W2, b1, deg):
    return pl.pallas_call(
        _tc2_body,
        grid=(GRID,),
        in_specs=[
            pl.BlockSpec((2, BLK, 128), lambda i: (0, i, 0)),
            pl.BlockSpec((128, 64), lambda i: (0, 0)),
            pl.BlockSpec((1, 128), lambda i: (0, 0)),
            pl.BlockSpec((2, BLK, 8), lambda i: (0, i, 0)),
        ],
        out_specs=pl.BlockSpec((BLK, 128), lambda i: (i, 0)),
        out_shape=jax.ShapeDtypeStruct((N_PAD, 128), jnp.float32),
    )(s1, W2, b1, deg)


def _tc3_body(s_ref, b_ref, deg_ref, z_ref):
    deg = deg_ref[0, :, 0] + deg_ref[1, :, 0] + 1.0
    dinv = lax.rsqrt(deg)
    z = (s_ref[0, :, :64] + s_ref[1, :, :64]) * dinv[:, None]
    z_ref[...] = z + b_ref[0][None, :]


def _tc3(s2, b2, deg):
    return pl.pallas_call(
        _tc3_body,
        grid=(GRID,),
        in_specs=[
            pl.BlockSpec((2, BLK, 128), lambda i: (0, i, 0)),
            pl.BlockSpec((1, 64), lambda i: (0, 0)),
            pl.BlockSpec((2, BLK, 8), lambda i: (0, i, 0)),
        ],
        out_specs=pl.BlockSpec((BLK, 64), lambda i: (i, 0)),
        out_shape=jax.ShapeDtypeStruct((N_PAD, 64), jnp.float32),
    )(s2, b2, deg)


def kernel(x, edge_index, W1, b1, W2, b2):
    ei = edge_index.astype(jnp.int32)
    pad = E_PAD - E_EDGES
    pad_rows = N_NODES + jnp.arange(pad, dtype=jnp.int32) % (N_PAD - N_NODES)
    src = jnp.concatenate([ei[0], pad_rows])
    dst = jnp.concatenate([ei[1], pad_rows])
    src32 = src.reshape(NC * NS, NCH, CHUNK)
    dst32 = dst.reshape(NC * NS, NCH, CHUNK)
    dst_deg = dst.reshape(NC * NS, NCH, CHUNK)

    ones128 = jnp.ones((CHUNK, 128), jnp.float32)
    zero128 = jnp.zeros((RPT, 128), jnp.float32)
    x_pad = jnp.pad(x, ((0, N_PAD - N_NODES), (0, 0)))

    deg_parts = _deg_kernel(dst_deg, ones128, zero128)
    deg = deg_parts.reshape(NC, N_PAD, 128)[:, :, :8]

    g1 = _tc1(x_pad, W1, deg)
    s1 = _prop(g1, zero128, src32, dst32).reshape(NC, N_PAD, 128)

    g2 = _tc2(s1, W2, b1.reshape(1, 128), deg)
    s2 = _prop(g2, zero128, src32, dst32).reshape(NC, N_PAD, 128)

    z = _tc3(s2, b2.reshape(1, 64), deg)
    return z[:N_NODES]
